# baseline (speedup 1.0000x reference)
import jax
import jax.numpy as jnp
from jax import lax
from jax.experimental import pallas as pl
from jax.experimental.pallas import tpu as pltpu
from jax.experimental.pallas import tpu_sc as plsc

N = 10000
D = 128
HID = 125

NC = 2
NS = 16
L = 16
NB = 3

G = 159
EPT = G * 128
EPAD = NS * EPT
ROWS_A = 624
ROWS_B = N - 15 * ROWS_A

_SPLAT_DNUMS = lax.GatherDimensionNumbers(
    offset_dims=(), collapsed_slice_dims=(0,), start_index_map=(0,))


def _splat(vec, t):
    return lax.gather(vec, jnp.full((L, 1), t, jnp.int32), _SPLAT_DNUMS,
                      (1,), mode=lax.GatherScatterMode.PROMISE_IN_BOUNDS)


def _sc_body(x_hbm, meta_hbm, w_hbm, zeros_hbm, out_hbm,
             meta0, meta1, meta2, wv0, wv1, wv2, sct0, sct1, sct2,
             rows0, rows1, rows2, acc_sh,
             im0, im1, im2, iw0, iw1, iw2, gm0, gm1, gm2, sm0, sm1, sm2):
    cid = lax.axis_index("c")
    sid = lax.axis_index("s")
    metas = (meta0, meta1, meta2)
    wvs = (wv0, wv1, wv2)
    scts = (sct0, sct1, sct2)
    rows = (rows0, rows1, rows2)
    ims = (im0, im1, im2)
    iws = (iw0, iw1, iw2)
    gms = (gm0, gm1, gm2)
    sms = (sm0, sm1, sm2)

    def gather_of(s):
        return pltpu.make_async_copy(x_hbm.at[metas[s].at[0]], rows[s], gms[s])

    def scatter_of(s):
        return pltpu.make_async_copy(rows[s], acc_sh.at[scts[s].at[0]], sms[s])

    def meta_of(s, g):
        return pltpu.make_async_copy(meta_hbm.at[cid, sid, g], metas[s], ims[s])

    def w_of(s, g):
        return pltpu.make_async_copy(w_hbm.at[sid, g], wvs[s], iws[s])

    for s in range(NB):
        meta_of(s, s).start()
        w_of(s, s).start()

    @pl.when(sid < 15)
    def _():
        pltpu.sync_copy(zeros_hbm.at[pl.ds(0, ROWS_A)],
                        acc_sh.at[pl.ds(sid * ROWS_A, ROWS_A)])

    @pl.when(sid == 15)
    def _():
        pltpu.sync_copy(zeros_hbm, acc_sh.at[pl.ds(15 * ROWS_A, ROWS_B)])

    plsc.subcore_barrier()
    meta_of(0, 0).wait()
    gather_of(0).start()
    meta_of(1, 1).wait()
    gather_of(1).start()

    def triple(p, carry):
        for b in range(NB):
            bn = (b + 1) % NB
            bnn = (b + 2) % NB
            g = NB * p + b
            m = metas[b]
            r = rows[b]

            @pl.when(g >= 1)
            def _():
                scatter_of(bnn).wait()

            @pl.when(g + 2 < G)
            def _():
                meta_of(bnn, g + 2).wait()
                gather_of(bnn).start()

            gather_of(b).wait()
            w_of(b, g).wait()

            def sixteen(s, c):
                w16 = wvs[b][0, pl.ds(s * L, L)]
                sct_sl = pl.ds(s * L, L)
                scts[b][0, sct_sl] = m[1, sct_sl]
                for t in range(L):
                    wr = _splat(w16, t)
                    row = s * L + t
                    for k in range(D // L):
                        sl = pl.ds(k * L, L)
                        r[row, sl] = r[row, sl] * wr
                return c

            lax.fori_loop(0, 128 // L, sixteen, 0)

            pltpu.async_copy(r, acc_sh.at[scts[b].at[0]], sms[b], add=True)

            @pl.when(g + NB < G)
            def _():
                meta_of(b, g + NB).start()
                w_of(b, g + NB).start()

        return carry

    lax.fori_loop(0, G // NB, triple, 0)

    scatter_of((G - 1) % NB).wait()
    plsc.subcore_barrier()

    @pl.when(sid < 15)
    def _():
        sl = pl.ds(sid * ROWS_A, ROWS_A)
        pltpu.sync_copy(acc_sh.at[sl], out_hbm.at[cid, sl])

    @pl.when(sid == 15)
    def _():
        sl = pl.ds(15 * ROWS_A, ROWS_B)
        pltpu.sync_copy(acc_sh.at[sl], out_hbm.at[cid, sl])


@jax.jit
def _segment_sums(X, w, Ri, Ro):
    pad = EPAD - Ri.shape[0]
    w_p = jnp.pad(w, (0, pad))
    w_t = w_p.reshape(NS, G, 1, 128)
    ri = jnp.pad(Ri, (0, pad)).astype(jnp.int32).reshape(NS, G, 128)
    ro = jnp.pad(Ro, (0, pad)).astype(jnp.int32).reshape(NS, G, 128)
    toff = (jnp.arange(NS, dtype=jnp.int32) * N).reshape(NS, 1, 1)
    m0 = jnp.stack([ro + toff, ri], axis=2)
    m1 = jnp.stack([ri + 16 * N + toff, ro], axis=2)
    meta = jnp.stack([m0, m1])
    zeros = jnp.zeros((ROWS_B, D), jnp.float32)

    mesh = plsc.VectorSubcoreMesh(core_axis_name="c", subcore_axis_name="s")
    f = pl.kernel(
        _sc_body,
        out_type=jax.ShapeDtypeStruct((NC, N, D), jnp.float32),
        mesh=mesh,
        scratch_types=(
            [pltpu.VMEM((2, 128), jnp.int32)] * NB
            + [pltpu.VMEM((1, 128), jnp.float32)] * NB
            + [pltpu.VMEM((1, 128), jnp.int32)] * NB
            + [pltpu.VMEM((128, D), jnp.float32)] * NB
            + [pltpu.VMEM_SHARED((N, D), jnp.float32)]
            + [pltpu.SemaphoreType.DMA] * (4 * NB)
        ),
    )
    return f(jnp.tile(X, (32, 1)), meta, w_t, zeros)


def _mlp_body(mimo_ref, x_ref, w1_ref, b1_ref, w2_ref, b2_ref, out_ref):
    mi = mimo_ref[0]
    mo = mimo_ref[1]
    x = x_ref[...]
    acc = jnp.dot(mi, w1_ref[0:D, :], preferred_element_type=jnp.float32)
    acc += jnp.dot(mo, w1_ref[D:2 * D, :], preferred_element_type=jnp.float32)
    acc += jnp.dot(x, w1_ref[2 * D:3 * D, :], preferred_element_type=jnp.float32)
    h = jnp.tanh(acc + b1_ref[...])
    out = jnp.tanh(jnp.dot(h, w2_ref[...], preferred_element_type=jnp.float32)
                   + b2_ref[...])
    out_ref[...] = out


def _mlp(mimo, X, W1, b1, W2, b2):
    R = 2000
    grid = (N // R,)
    return pl.pallas_call(
        _mlp_body,
        grid=grid,
        in_specs=[
            pl.BlockSpec((NC, R, D), lambda i: (0, i, 0)),
            pl.BlockSpec((R, D), lambda i: (i, 0)),
            pl.BlockSpec((3 * D, HID), lambda i: (0, 0)),
            pl.BlockSpec((1, HID), lambda i: (0, 0)),
            pl.BlockSpec((HID, HID), lambda i: (0, 0)),
            pl.BlockSpec((1, HID), lambda i: (0, 0)),
        ],
        out_specs=pl.BlockSpec((R, HID), lambda i: (i, 0)),
        out_shape=jax.ShapeDtypeStruct((N, HID), jnp.float32),
    )(mimo, X, W1, b1, W2, b2)


def kernel(X, e, Ri, Ro, W1, b1, W2, b2):
    w = e[:, 0]
    mimo = _segment_sums(X, w, Ri, Ro)
    return _mlp(mimo, X, W1, b1.reshape(1, HID), W2, b2.reshape(1, HID))


# speedup vs baseline: 1.0835x; 1.0835x over previous
import jax
import jax.numpy as jnp
from jax import lax
from jax.experimental import pallas as pl
from jax.experimental.pallas import tpu as pltpu
from jax.experimental.pallas import tpu_sc as plsc

N = 10000
D = 128
HID = 125

NC = 2
NS = 16
L = 16
NB = 3

G = 159
EPT = G * 128
EPAD = NS * EPT
ROWS_A = 624
ROWS_B = N - 15 * ROWS_A

_SPLAT_DNUMS = lax.GatherDimensionNumbers(
    offset_dims=(), collapsed_slice_dims=(0,), start_index_map=(0,))


def _splat(vec, t):
    return lax.gather(vec, jnp.full((L, 1), t, jnp.int32), _SPLAT_DNUMS,
                      (1,), mode=lax.GatherScatterMode.PROMISE_IN_BOUNDS)


def _sc_body(x_hbm, meta_hbm, w_hbm, zeros_hbm, out_hbm,
             meta0, meta1, meta2, wv0, wv1, wv2, sct0, sct1, sct2,
             rows0, rows1, rows2, acc_sh,
             im0, im1, im2, iw0, iw1, iw2, gm0, gm1, gm2, sm0, sm1, sm2):
    cid = lax.axis_index("c")
    sid = lax.axis_index("s")
    metas = (meta0, meta1, meta2)
    wvs = (wv0, wv1, wv2)
    scts = (sct0, sct1, sct2)
    rows = (rows0, rows1, rows2)
    ims = (im0, im1, im2)
    iws = (iw0, iw1, iw2)
    gms = (gm0, gm1, gm2)
    sms = (sm0, sm1, sm2)

    def gather_of(s):
        return pltpu.make_async_copy(x_hbm.at[metas[s].at[0]], rows[s], gms[s])

    def scatter_of(s):
        return pltpu.make_async_copy(rows[s], acc_sh.at[scts[s].at[0]], sms[s])

    def meta_of(s, g):
        return pltpu.make_async_copy(meta_hbm.at[cid, sid, g], metas[s], ims[s])

    def w_of(s, g):
        return pltpu.make_async_copy(w_hbm.at[sid, g], wvs[s], iws[s])

    for s in range(NB):
        meta_of(s, s).start()
        w_of(s, s).start()

    @pl.when(sid < 15)
    def _():
        pltpu.sync_copy(zeros_hbm.at[pl.ds(0, ROWS_A)],
                        acc_sh.at[pl.ds(sid * ROWS_A, ROWS_A)])

    @pl.when(sid == 15)
    def _():
        pltpu.sync_copy(zeros_hbm, acc_sh.at[pl.ds(15 * ROWS_A, ROWS_B)])

    plsc.subcore_barrier()
    meta_of(0, 0).wait()
    gather_of(0).start()
    meta_of(1, 1).wait()
    gather_of(1).start()

    def triple(p, carry):
        for b in range(NB):
            bn = (b + 1) % NB
            bnn = (b + 2) % NB
            g = NB * p + b
            m = metas[b]
            r = rows[b]

            @pl.when(g >= 1)
            def _():
                scatter_of(bnn).wait()

            @pl.when(g + 2 < G)
            def _():
                meta_of(bnn, g + 2).wait()
                gather_of(bnn).start()

            gather_of(b).wait()
            w_of(b, g).wait()

            def sixteen(s, c):
                w16 = wvs[b][0, pl.ds(s * L, L)]
                sct_sl = pl.ds(s * L, L)
                scts[b][0, sct_sl] = m[1, sct_sl]
                for t in range(L):
                    wr = _splat(w16, t)
                    row = s * L + t
                    for k in range(D // L):
                        sl = pl.ds(k * L, L)
                        r[row, sl] = r[row, sl] * wr
                return c

            lax.fori_loop(0, 128 // L, sixteen, 0)

            pltpu.async_copy(r, acc_sh.at[scts[b].at[0]], sms[b], add=True)

            @pl.when(g + NB < G)
            def _():
                meta_of(b, g + NB).start()
                w_of(b, g + NB).start()

        return carry

    lax.fori_loop(0, G // NB, triple, 0)

    scatter_of((G - 1) % NB).wait()
    plsc.subcore_barrier()

    @pl.when(sid < 15)
    def _():
        sl = pl.ds(sid * ROWS_A, ROWS_A)
        pltpu.sync_copy(acc_sh.at[sl], out_hbm.at[cid, sl])

    @pl.when(sid == 15)
    def _():
        sl = pl.ds(15 * ROWS_A, ROWS_B)
        pltpu.sync_copy(acc_sh.at[sl], out_hbm.at[cid, sl])


@jax.jit
def _segment_sums(X, w, Ri, Ro):
    pad = EPAD - Ri.shape[0]
    w_p = jnp.pad(w, (0, pad))
    w_t = w_p.reshape(NS, G, 1, 128)
    ri = jnp.pad(Ri, (0, pad)).astype(jnp.int32).reshape(NS, G, 128)
    ro = jnp.pad(Ro, (0, pad)).astype(jnp.int32).reshape(NS, G, 128)
    toff = (jnp.arange(NS, dtype=jnp.int32) // 2 * N).reshape(NS, 1, 1)
    m0 = jnp.stack([ro + toff, ri], axis=2)
    m1 = jnp.stack([ri + 8 * N + toff, ro], axis=2)
    meta = jnp.stack([m0, m1])
    zeros = jnp.zeros((ROWS_B, D), jnp.float32)

    mesh = plsc.VectorSubcoreMesh(core_axis_name="c", subcore_axis_name="s")
    f = pl.kernel(
        _sc_body,
        out_type=jax.ShapeDtypeStruct((NC, N, D), jnp.float32),
        mesh=mesh,
        scratch_types=(
            [pltpu.VMEM((2, 128), jnp.int32)] * NB
            + [pltpu.VMEM((1, 128), jnp.float32)] * NB
            + [pltpu.VMEM((1, 128), jnp.int32)] * NB
            + [pltpu.VMEM((128, D), jnp.float32)] * NB
            + [pltpu.VMEM_SHARED((N, D), jnp.float32)]
            + [pltpu.SemaphoreType.DMA] * (4 * NB)
        ),
    )
    return f(jnp.tile(X, (16, 1)), meta, w_t, zeros)


def _mlp_body(mimo_ref, x_ref, w1_ref, b1_ref, w2_ref, b2_ref, out_ref):
    mi = mimo_ref[0]
    mo = mimo_ref[1]
    x = x_ref[...]
    acc = jnp.dot(mi, w1_ref[0:D, :], preferred_element_type=jnp.float32)
    acc += jnp.dot(mo, w1_ref[D:2 * D, :], preferred_element_type=jnp.float32)
    acc += jnp.dot(x, w1_ref[2 * D:3 * D, :], preferred_element_type=jnp.float32)
    h = jnp.tanh(acc + b1_ref[...])
    out = jnp.tanh(jnp.dot(h, w2_ref[...], preferred_element_type=jnp.float32)
                   + b2_ref[...])
    out_ref[...] = out


def _mlp(mimo, X, W1, b1, W2, b2):
    R = 2000
    grid = (N // R,)
    return pl.pallas_call(
        _mlp_body,
        grid=grid,
        in_specs=[
            pl.BlockSpec((NC, R, D), lambda i: (0, i, 0)),
            pl.BlockSpec((R, D), lambda i: (i, 0)),
            pl.BlockSpec((3 * D, HID), lambda i: (0, 0)),
            pl.BlockSpec((1, HID), lambda i: (0, 0)),
            pl.BlockSpec((HID, HID), lambda i: (0, 0)),
            pl.BlockSpec((1, HID), lambda i: (0, 0)),
        ],
        out_specs=pl.BlockSpec((R, HID), lambda i: (i, 0)),
        out_shape=jax.ShapeDtypeStruct((N, HID), jnp.float32),
    )(mimo, X, W1, b1, W2, b2)


def kernel(X, e, Ri, Ro, W1, b1, W2, b2):
    w = e[:, 0]
    mimo = _segment_sums(X, w, Ri, Ro)
    return _mlp(mimo, X, W1, b1.reshape(1, HID), W2, b2.reshape(1, HID))
